# Initial kernel scaffold; baseline (speedup 1.0000x reference)
import jax
import jax.numpy as jnp
from jax.experimental import pallas as pl


def kernel(x, edge_index, W):
    raise NotImplementedError("write your pallas kernel here")


# speedup vs baseline: 110.1453x; 110.1453x over previous
import functools

import jax
import jax.numpy as jnp
from jax import lax
from jax.experimental import pallas as pl
from jax.experimental.pallas import tpu as pltpu
from jax.experimental.pallas import tpu_sc as plsc

N_NODES = 10000
N_EDGES = 320000
D_FEAT = 128

NC = 2
NS = 16
LANES = 128
NW = NC * NS
ROWS = 79
NPAD = ROWS * LANES
EPAD = NW * ROWS * LANES
PAD_IDX = N_NODES


def _worker_id():
    return lax.axis_index("s") * NC + lax.axis_index("c")


@functools.partial(
    pl.kernel,
    out_type=jax.ShapeDtypeStruct((NC, NPAD), jnp.float32),
    mesh=plsc.VectorSubcoreMesh(core_axis_name="c", subcore_axis_name="s"),
    compiler_params=pltpu.CompilerParams(use_tc_tiling_on_sc=False, needs_layout_passes=False),
    scratch_types=[
        pltpu.VMEM((ROWS, LANES), jnp.int32),
        pltpu.VMEM((LANES,), jnp.float32),
        pltpu.VMEM_SHARED((NPAD,), jnp.float32),
    ],
)
def _hist_kernel(dst_hbm, zeros_hbm, out_hbm, dst_v, ones_v, acc_sh):
    cid = lax.axis_index("c")
    sid = lax.axis_index("s")
    wid = _worker_id()

    @pl.when(sid == 0)
    def _():
        pltpu.sync_copy(zeros_hbm, acc_sh)

    pltpu.sync_copy(dst_hbm.at[wid], dst_v)
    for t in range(LANES // 16):
        ones_v[pl.ds(t * 16, 16)] = jnp.ones((16,), jnp.float32)

    plsc.subcore_barrier()

    def body(j, carry):
        pltpu.sync_copy(ones_v, acc_sh.at[dst_v.at[j]], add=True)
        return carry

    lax.fori_loop(0, ROWS, body, 0)
    plsc.subcore_barrier()

    @pl.when(sid == 0)
    def _():
        pltpu.sync_copy(acc_sh, out_hbm.at[cid])


@functools.partial(
    pl.kernel,
    out_type=jax.ShapeDtypeStruct((NC, NPAD), jnp.float32),
    mesh=plsc.VectorSubcoreMesh(core_axis_name="c", subcore_axis_name="s"),
    compiler_params=pltpu.CompilerParams(use_tc_tiling_on_sc=False, needs_layout_passes=False),
    scratch_types=[
        pltpu.VMEM((NPAD,), jnp.float32),
        pltpu.VMEM((ROWS, LANES), jnp.int32),
        pltpu.VMEM((ROWS, LANES), jnp.int32),
        pltpu.VMEM((ROWS, LANES), jnp.float32),
        pltpu.VMEM_SHARED((NPAD,), jnp.float32),
    ],
)
def _msg_kernel(src_hbm, dst_hbm, g_hbm, zeros_hbm, out_hbm,
                g_v, src_v, dst_v, vals_v, acc_sh):
    cid = lax.axis_index("c")
    sid = lax.axis_index("s")
    wid = _worker_id()

    @pl.when(sid == 0)
    def _():
        pltpu.sync_copy(zeros_hbm, acc_sh)

    pltpu.sync_copy(g_hbm, g_v)
    pltpu.sync_copy(src_hbm.at[wid], src_v)
    pltpu.sync_copy(dst_hbm.at[wid], dst_v)

    def gather_row(j, carry):
        for t in range(LANES // 16):
            idx = src_v[j, pl.ds(t * 16, 16)]
            vals_v[j, pl.ds(t * 16, 16)] = plsc.load_gather(g_v, [idx])
        return carry

    lax.fori_loop(0, ROWS, gather_row, 0)
    plsc.subcore_barrier()

    def scatter_row(j, carry):
        pltpu.sync_copy(vals_v.at[j], acc_sh.at[dst_v.at[j]], add=True)
        return carry

    lax.fori_loop(0, ROWS, scatter_row, 0)
    plsc.subcore_barrier()

    @pl.when(sid == 0)
    def _():
        pltpu.sync_copy(acc_sh, out_hbm.at[cid])


def _dense_body(x3_ref, w_ref, deg2_ref, g_ref, dis_ref):
    h = jnp.sum(x3_ref[...] * w_ref[...], axis=2)
    deg = deg2_ref[0] + deg2_ref[1] + 1.0
    dis = lax.rsqrt(deg)
    node = (lax.broadcasted_iota(jnp.int32, (ROWS, LANES), 0) * LANES
            + lax.broadcasted_iota(jnp.int32, (ROWS, LANES), 1))
    g_ref[...] = jnp.where(node < N_NODES, dis * h, 0.0)
    dis_ref[...] = dis


_dense_call = pl.pallas_call(
    _dense_body,
    out_shape=[
        jax.ShapeDtypeStruct((ROWS, LANES), jnp.float32),
        jax.ShapeDtypeStruct((ROWS, LANES), jnp.float32),
    ],
)


def _combine_body(acc2_ref, g_ref, dis_ref, out_ref):
    out_ref[...] = dis_ref[...] * (g_ref[...] + acc2_ref[0] + acc2_ref[1])


_combine_call = pl.pallas_call(
    _combine_body,
    out_shape=jax.ShapeDtypeStruct((ROWS, LANES), jnp.float32),
)


def kernel(x, edge_index, W):
    src = edge_index[0].astype(jnp.int32)
    dst = edge_index[1].astype(jnp.int32)
    pad = jnp.full((EPAD - N_EDGES,), PAD_IDX, jnp.int32)
    src_p = jnp.concatenate([src, pad]).reshape(NW, ROWS, LANES)
    dst_p = jnp.concatenate([dst, pad]).reshape(NW, ROWS, LANES)

    x3 = jnp.pad(x, ((0, NPAD - N_NODES), (0, 0))).reshape(ROWS, LANES, D_FEAT)
    w3 = W.reshape(1, 1, D_FEAT)
    zeros = jnp.zeros((NPAD,), jnp.float32)

    deg2 = _hist_kernel(dst_p, zeros)
    g2d, dis2d = _dense_call(x3, w3, deg2.reshape(NC, ROWS, LANES))
    acc2 = _msg_kernel(src_p, dst_p, g2d.reshape(NPAD), zeros)
    out2d = _combine_call(acc2.reshape(NC, ROWS, LANES), g2d, dis2d)
    return out2d.reshape(NPAD)[:N_NODES]


# speedup vs baseline: 142.6437x; 1.2951x over previous
import functools

import jax
import jax.numpy as jnp
from jax import lax
from jax.experimental import pallas as pl
from jax.experimental.pallas import tpu as pltpu
from jax.experimental.pallas import tpu_sc as plsc

N_NODES = 10000
N_EDGES = 320000
D_FEAT = 128

NC = 2
NS = 16
LANES = 128
NW = NC * NS
EROWS = N_EDGES // LANES
WROWS = 79
LAST_W = NW - 1
LAST_ROW0 = LAST_W * WROWS
LAST_NROWS = EROWS - LAST_ROW0
NR = 625

_SC_PARAMS = pltpu.CompilerParams(
    use_tc_tiling_on_sc=False, needs_layout_passes=False)


def _ids():
    cid = lax.axis_index("c")
    sid = lax.axis_index("s")
    return cid, sid, sid * NC + cid


@functools.partial(
    pl.kernel,
    out_type=jax.ShapeDtypeStruct((NC, N_NODES), jnp.float32),
    mesh=plsc.VectorSubcoreMesh(core_axis_name="c", subcore_axis_name="s"),
    compiler_params=_SC_PARAMS,
    scratch_types=[
        pltpu.VMEM((WROWS, LANES), jnp.int32),
        pltpu.VMEM((LANES,), jnp.float32),
        pltpu.VMEM_SHARED((N_NODES,), jnp.float32),
        pltpu.SemaphoreType.DMA,
    ],
)
def _hist_kernel(edges_hbm, zeros_hbm, out_hbm, dst_v, ones_v, acc_sh, sem):
    cid, sid, wid = _ids()

    @pl.when(sid == 0)
    def _():
        pltpu.sync_copy(zeros_hbm, acc_sh)

    @pl.when(wid < LAST_W)
    def _():
        pltpu.sync_copy(edges_hbm.at[1, pl.ds(wid * WROWS, WROWS)], dst_v)

    @pl.when(wid == LAST_W)
    def _():
        pltpu.sync_copy(edges_hbm.at[1, pl.ds(LAST_ROW0, LAST_NROWS)],
                        dst_v.at[pl.ds(0, LAST_NROWS)])

    for t in range(LANES // 16):
        ones_v[pl.ds(t * 16, 16)] = jnp.ones((16,), jnp.float32)

    nrows = jnp.where(wid == LAST_W, LAST_NROWS, WROWS)
    plsc.subcore_barrier()

    def issue(j, carry):
        pltpu.async_copy(ones_v, acc_sh.at[dst_v.at[j]], sem, add=True)
        return carry

    lax.fori_loop(0, nrows, issue, 0)

    def drain(j, carry):
        pltpu.make_async_copy(ones_v, acc_sh.at[dst_v.at[0]], sem).wait()
        return carry

    lax.fori_loop(0, nrows, drain, 0)
    plsc.subcore_barrier()

    @pl.when(sid == 0)
    def _():
        pltpu.sync_copy(acc_sh, out_hbm.at[cid])


@functools.partial(
    pl.kernel,
    out_type=jax.ShapeDtypeStruct((NC, N_NODES), jnp.float32),
    mesh=plsc.VectorSubcoreMesh(core_axis_name="c", subcore_axis_name="s"),
    compiler_params=_SC_PARAMS,
    scratch_types=[
        pltpu.VMEM((N_NODES,), jnp.float32),
        pltpu.VMEM((WROWS, LANES), jnp.int32),
        pltpu.VMEM((WROWS, LANES), jnp.int32),
        pltpu.VMEM((WROWS, LANES), jnp.float32),
        pltpu.VMEM_SHARED((N_NODES,), jnp.float32),
        pltpu.SemaphoreType.DMA,
    ],
)
def _msg_kernel(edges_hbm, g_hbm, zeros_hbm, out_hbm,
                g_v, src_v, dst_v, vals_v, acc_sh, sem):
    cid, sid, wid = _ids()

    @pl.when(sid == 0)
    def _():
        pltpu.sync_copy(zeros_hbm, acc_sh)

    pltpu.sync_copy(g_hbm, g_v)

    @pl.when(wid < LAST_W)
    def _():
        pltpu.sync_copy(edges_hbm.at[0, pl.ds(wid * WROWS, WROWS)], src_v)
        pltpu.sync_copy(edges_hbm.at[1, pl.ds(wid * WROWS, WROWS)], dst_v)

    @pl.when(wid == LAST_W)
    def _():
        pltpu.sync_copy(edges_hbm.at[0, pl.ds(LAST_ROW0, LAST_NROWS)],
                        src_v.at[pl.ds(0, LAST_NROWS)])
        pltpu.sync_copy(edges_hbm.at[1, pl.ds(LAST_ROW0, LAST_NROWS)],
                        dst_v.at[pl.ds(0, LAST_NROWS)])

    nrows = jnp.where(wid == LAST_W, LAST_NROWS, WROWS)
    plsc.subcore_barrier()

    def row_fn(j, carry):
        for t in range(LANES // 16):
            idx = src_v[j, pl.ds(t * 16, 16)]
            vals_v[j, pl.ds(t * 16, 16)] = plsc.load_gather(g_v, [idx])
        pltpu.async_copy(vals_v.at[j], acc_sh.at[dst_v.at[j]], sem, add=True)
        return carry

    lax.fori_loop(0, nrows, row_fn, 0)

    def drain(j, carry):
        pltpu.make_async_copy(vals_v.at[0], acc_sh.at[dst_v.at[0]], sem).wait()
        return carry

    lax.fori_loop(0, nrows, drain, 0)
    plsc.subcore_barrier()

    @pl.when(sid == 0)
    def _():
        pltpu.sync_copy(acc_sh, out_hbm.at[cid])


def _dense_body(x3_ref, w_ref, deg2_ref, g_ref, dis_ref):
    h = jnp.sum(x3_ref[...] * w_ref[...], axis=2)
    deg = deg2_ref[0] + deg2_ref[1] + 1.0
    dis = lax.rsqrt(deg)
    g_ref[...] = dis * h
    dis_ref[...] = dis


_dense_call = pl.pallas_call(
    _dense_body,
    out_shape=[
        jax.ShapeDtypeStruct((NR, 16), jnp.float32),
        jax.ShapeDtypeStruct((NR, 16), jnp.float32),
    ],
)


def _combine_body(acc2_ref, g_ref, dis_ref, out_ref):
    out_ref[...] = dis_ref[...] * (g_ref[...] + acc2_ref[0] + acc2_ref[1])


_combine_call = pl.pallas_call(
    _combine_body,
    out_shape=jax.ShapeDtypeStruct((NR, 16), jnp.float32),
)


def kernel(x, edge_index, W):
    edges = edge_index.astype(jnp.int32).reshape(2, EROWS, LANES)
    x3 = x.reshape(NR, 16, D_FEAT)
    w3 = W.reshape(1, 1, D_FEAT)
    zeros = jnp.zeros((N_NODES,), jnp.float32)

    deg2 = _hist_kernel(edges, zeros)
    g2d, dis2d = _dense_call(x3, w3, deg2.reshape(NC, NR, 16))
    acc2 = _msg_kernel(edges, g2d.reshape(N_NODES), zeros)
    out2d = _combine_call(acc2.reshape(NC, NR, 16), g2d, dis2d)
    return out2d.reshape(N_NODES)


# speedup vs baseline: 152.2060x; 1.0670x over previous
import functools

import jax
import jax.numpy as jnp
from jax import lax
from jax.experimental import pallas as pl
from jax.experimental.pallas import tpu as pltpu
from jax.experimental.pallas import tpu_sc as plsc

N_NODES = 10000
N_EDGES = 320000
D_FEAT = 128

NC = 2
NS = 16
LANES = 128
NW = NC * NS
EROWS = N_EDGES // LANES
WROWS = EROWS // NW
XTRA = EROWS - WROWS * NW
NR = 625
ACC = 10240
ZCHUNK = ACC // NS

_SC_PARAMS = pltpu.CompilerParams(
    use_tc_tiling_on_sc=False, needs_layout_passes=False)


def _ids():
    cid = lax.axis_index("c")
    sid = lax.axis_index("s")
    return cid, sid, sid * NC + cid


def _row0(wid):
    return jnp.where(wid < XTRA, (WROWS + 1) * wid, WROWS * wid + XTRA)


def _zero_init(zbuf, acc_sh, sid):
    for t in range(ZCHUNK // 16):
        zbuf[pl.ds(t * 16, 16)] = jnp.zeros((16,), jnp.float32)
    pltpu.sync_copy(zbuf, acc_sh.at[pl.ds(sid * ZCHUNK, ZCHUNK)])


@functools.partial(
    pl.kernel,
    out_type=jax.ShapeDtypeStruct((NC, N_NODES), jnp.float32),
    mesh=plsc.VectorSubcoreMesh(core_axis_name="c", subcore_axis_name="s"),
    compiler_params=_SC_PARAMS,
    scratch_types=[
        pltpu.VMEM((WROWS + 1, LANES), jnp.int32),
        pltpu.VMEM((LANES,), jnp.float32),
        pltpu.VMEM((ZCHUNK,), jnp.float32),
        pltpu.VMEM_SHARED((ACC,), jnp.float32),
        pltpu.SemaphoreType.DMA,
    ],
)
def _hist_kernel(edges_hbm, out_hbm, dst_v, ones_v, zbuf, acc_sh, sem):
    cid, sid, wid = _ids()
    row0 = _row0(wid)

    pltpu.sync_copy(edges_hbm.at[1, pl.ds(row0, WROWS)],
                    dst_v.at[pl.ds(0, WROWS)])

    @pl.when(wid < XTRA)
    def _():
        pltpu.sync_copy(edges_hbm.at[1, pl.ds(row0 + WROWS, 1)],
                        dst_v.at[pl.ds(WROWS, 1)])

    for t in range(LANES // 16):
        ones_v[pl.ds(t * 16, 16)] = jnp.ones((16,), jnp.float32)

    _zero_init(zbuf, acc_sh, sid)
    plsc.subcore_barrier()

    def issue(j, carry):
        pltpu.async_copy(ones_v, acc_sh.at[dst_v.at[j]], sem, add=True)
        return carry

    lax.fori_loop(0, WROWS, issue, 0)

    @pl.when(wid < XTRA)
    def _():
        issue(WROWS, 0)

    def drain(j, carry):
        pltpu.make_async_copy(ones_v, acc_sh.at[dst_v.at[0]], sem).wait()
        return carry

    lax.fori_loop(0, WROWS, drain, 0)

    @pl.when(wid < XTRA)
    def _():
        drain(0, 0)

    plsc.subcore_barrier()

    @pl.when(sid == 0)
    def _():
        pltpu.sync_copy(acc_sh.at[pl.ds(0, N_NODES)], out_hbm.at[cid])


@functools.partial(
    pl.kernel,
    out_type=jax.ShapeDtypeStruct((NC, N_NODES), jnp.float32),
    mesh=plsc.VectorSubcoreMesh(core_axis_name="c", subcore_axis_name="s"),
    compiler_params=_SC_PARAMS,
    scratch_types=[
        pltpu.VMEM((N_NODES,), jnp.float32),
        pltpu.VMEM((WROWS + 1, LANES), jnp.int32),
        pltpu.VMEM((WROWS + 1, LANES), jnp.int32),
        pltpu.VMEM((WROWS + 1, LANES), jnp.float32),
        pltpu.VMEM((ZCHUNK,), jnp.float32),
        pltpu.VMEM_SHARED((ACC,), jnp.float32),
        pltpu.SemaphoreType.DMA,
        pltpu.SemaphoreType.DMA,
    ],
)
def _msg_kernel(edges_hbm, g_hbm, out_hbm,
                g_v, src_v, dst_v, vals_v, zbuf, acc_sh, sem, gsem):
    cid, sid, wid = _ids()
    row0 = _row0(wid)

    gcopy = pltpu.async_copy(g_hbm, g_v, gsem)
    pltpu.sync_copy(edges_hbm.at[0, pl.ds(row0, WROWS)],
                    src_v.at[pl.ds(0, WROWS)])
    pltpu.sync_copy(edges_hbm.at[1, pl.ds(row0, WROWS)],
                    dst_v.at[pl.ds(0, WROWS)])

    @pl.when(wid < XTRA)
    def _():
        pltpu.sync_copy(edges_hbm.at[0, pl.ds(row0 + WROWS, 1)],
                        src_v.at[pl.ds(WROWS, 1)])
        pltpu.sync_copy(edges_hbm.at[1, pl.ds(row0 + WROWS, 1)],
                        dst_v.at[pl.ds(WROWS, 1)])

    _zero_init(zbuf, acc_sh, sid)
    gcopy.wait()
    plsc.subcore_barrier()

    def row_fn(j, carry):
        for t in range(LANES // 16):
            idx = src_v[j, pl.ds(t * 16, 16)]
            vals_v[j, pl.ds(t * 16, 16)] = plsc.load_gather(g_v, [idx])
        pltpu.async_copy(vals_v.at[j], acc_sh.at[dst_v.at[j]], sem, add=True)
        return carry

    lax.fori_loop(0, WROWS, row_fn, 0)

    @pl.when(wid < XTRA)
    def _():
        row_fn(WROWS, 0)

    def drain(j, carry):
        pltpu.make_async_copy(vals_v.at[0], acc_sh.at[dst_v.at[0]], sem).wait()
        return carry

    lax.fori_loop(0, WROWS, drain, 0)

    @pl.when(wid < XTRA)
    def _():
        drain(0, 0)

    plsc.subcore_barrier()

    @pl.when(sid == 0)
    def _():
        pltpu.sync_copy(acc_sh.at[pl.ds(0, N_NODES)], out_hbm.at[cid])


def _dense_body(x3_ref, w_ref, deg2_ref, g_ref, dis_ref):
    h = jnp.sum(x3_ref[...] * w_ref[...], axis=2)
    deg = deg2_ref[0] + deg2_ref[1] + 1.0
    dis = lax.rsqrt(deg)
    g_ref[...] = dis * h
    dis_ref[...] = dis


_dense_call = pl.pallas_call(
    _dense_body,
    out_shape=[
        jax.ShapeDtypeStruct((NR, 16), jnp.float32),
        jax.ShapeDtypeStruct((NR, 16), jnp.float32),
    ],
)


def _combine_body(acc2_ref, g_ref, dis_ref, out_ref):
    out_ref[...] = dis_ref[...] * (g_ref[...] + acc2_ref[0] + acc2_ref[1])


_combine_call = pl.pallas_call(
    _combine_body,
    out_shape=jax.ShapeDtypeStruct((NR, 16), jnp.float32),
)


def kernel(x, edge_index, W):
    edges = edge_index.astype(jnp.int32).reshape(2, EROWS, LANES)
    x3 = x.reshape(NR, 16, D_FEAT)
    w3 = W.reshape(1, 1, D_FEAT)

    deg2 = _hist_kernel(edges)
    g2d, dis2d = _dense_call(x3, w3, deg2.reshape(NC, NR, 16))
    acc2 = _msg_kernel(edges, g2d.reshape(N_NODES))
    out2d = _combine_call(acc2.reshape(NC, NR, 16), g2d, dis2d)
    return out2d.reshape(N_NODES)


# speedup vs baseline: 166.6357x; 1.0948x over previous
import functools

import jax
import jax.numpy as jnp
from jax import lax
from jax.experimental import pallas as pl
from jax.experimental.pallas import tpu as pltpu
from jax.experimental.pallas import tpu_sc as plsc

N_NODES = 10000
N_EDGES = 320000
D_FEAT = 128

NC = 2
NS = 16
LANES = 128
NW = NC * NS
EROWS = N_EDGES // LANES
WROWS = EROWS // NW
XTRA = EROWS - WROWS * NW
WEDGE = (WROWS + 1) * LANES
NR = 625
ACC = 10240
ZCHUNK = ACC // NS
GTAIL = N_NODES - (NS - 1) * ZCHUNK

_SC_PARAMS = pltpu.CompilerParams(
    use_tc_tiling_on_sc=False, needs_layout_passes=False)


def _ids():
    cid = lax.axis_index("c")
    sid = lax.axis_index("s")
    return cid, sid, sid * NC + cid


def _row0(wid):
    return jnp.where(wid < XTRA, (WROWS + 1) * wid, WROWS * wid + XTRA)


def _zero_init(zbuf, acc_sh, sid):
    for t in range(ZCHUNK // 16):
        zbuf[pl.ds(t * 16, 16)] = jnp.zeros((16,), jnp.float32)
    pltpu.sync_copy(zbuf, acc_sh.at[pl.ds(sid * ZCHUNK, ZCHUNK)])


def _stage_edges(edges_hbm, comp, buf, row0):
    pltpu.sync_copy(edges_hbm.at[comp, pl.ds(row0 * LANES, WROWS * LANES)],
                    buf.at[pl.ds(0, WROWS * LANES)])


def _stage_edges_xtra(edges_hbm, comp, buf, row0):
    pltpu.sync_copy(edges_hbm.at[comp, pl.ds((row0 + WROWS) * LANES, LANES)],
                    buf.at[pl.ds(WROWS * LANES, LANES)])


def _rsqrt_nr(d):
    i = plsc.bitcast(d, jnp.int32)
    i = jnp.int32(0x5F3759DF) - lax.shift_right_arithmetic(i, 1)
    y = plsc.bitcast(i, jnp.float32)
    for _ in range(3):
        y = y * (1.5 - 0.5 * d * y * y)
    return y


@functools.partial(
    pl.kernel,
    out_type=jax.ShapeDtypeStruct((NC, N_NODES), jnp.float32),
    mesh=plsc.VectorSubcoreMesh(core_axis_name="c", subcore_axis_name="s"),
    compiler_params=_SC_PARAMS,
    scratch_types=[
        pltpu.VMEM((WEDGE,), jnp.int32),
        pltpu.VMEM((LANES,), jnp.float32),
        pltpu.VMEM((ZCHUNK,), jnp.float32),
        pltpu.VMEM_SHARED((ACC,), jnp.float32),
        pltpu.SemaphoreType.DMA,
    ],
)
def _hist_kernel(edges_hbm, out_hbm, dst_v, ones_v, zbuf, acc_sh, sem):
    cid, sid, wid = _ids()
    row0 = _row0(wid)

    _stage_edges(edges_hbm, 1, dst_v, row0)

    @pl.when(wid < XTRA)
    def _():
        _stage_edges_xtra(edges_hbm, 1, dst_v, row0)

    for t in range(LANES // 16):
        ones_v[pl.ds(t * 16, 16)] = jnp.ones((16,), jnp.float32)

    _zero_init(zbuf, acc_sh, sid)
    plsc.subcore_barrier()

    def issue(j, carry):
        pltpu.async_copy(ones_v, acc_sh.at[dst_v.at[pl.ds(j * LANES, LANES)]],
                         sem, add=True)
        return carry

    lax.fori_loop(0, WROWS, issue, 0)

    @pl.when(wid < XTRA)
    def _():
        issue(WROWS, 0)

    def drain(j, carry):
        pltpu.make_async_copy(ones_v, acc_sh.at[dst_v.at[pl.ds(0, LANES)]],
                              sem).wait()
        return carry

    lax.fori_loop(0, WROWS, drain, 0)

    @pl.when(wid < XTRA)
    def _():
        drain(0, 0)

    plsc.subcore_barrier()

    @pl.when(sid == 0)
    def _():
        pltpu.sync_copy(acc_sh.at[pl.ds(0, N_NODES)], out_hbm.at[cid])


@functools.partial(
    pl.kernel,
    out_type=jax.ShapeDtypeStruct((NC, N_NODES), jnp.float32),
    mesh=plsc.VectorSubcoreMesh(core_axis_name="c", subcore_axis_name="s"),
    compiler_params=_SC_PARAMS,
    scratch_types=[
        pltpu.VMEM((ACC,), jnp.float32),
        pltpu.VMEM((WEDGE,), jnp.int32),
        pltpu.VMEM((WEDGE,), jnp.int32),
        pltpu.VMEM((WEDGE,), jnp.float32),
        pltpu.VMEM((ZCHUNK,), jnp.float32),
        pltpu.VMEM((ZCHUNK,), jnp.float32),
        pltpu.VMEM((ZCHUNK,), jnp.float32),
        pltpu.VMEM_SHARED((ACC,), jnp.float32),
        pltpu.VMEM_SHARED((ACC,), jnp.float32),
        pltpu.SemaphoreType.DMA,
        pltpu.SemaphoreType.DMA,
    ],
)
def _msg_kernel(edges_hbm, deg2_hbm, h_hbm, out_hbm,
                g_v, src_v, dst_v, vals_v, d0_v, d1_v, zbuf,
                acc_sh, g_sh, sem, gsem):
    cid, sid, wid = _ids()
    row0 = _row0(wid)

    _stage_edges(edges_hbm, 0, src_v, row0)
    _stage_edges(edges_hbm, 1, dst_v, row0)

    @pl.when(wid < XTRA)
    def _():
        _stage_edges_xtra(edges_hbm, 0, src_v, row0)
        _stage_edges_xtra(edges_hbm, 1, dst_v, row0)

    node0 = sid * ZCHUNK
    nsz = jnp.where(sid == NS - 1, GTAIL, ZCHUNK)

    @pl.when(sid < NS - 1)
    def _():
        pltpu.sync_copy(deg2_hbm.at[0, pl.ds(node0, ZCHUNK)], d0_v)
        pltpu.sync_copy(deg2_hbm.at[1, pl.ds(node0, ZCHUNK)], d1_v)
        pltpu.sync_copy(h_hbm.at[pl.ds(node0, ZCHUNK)], zbuf)

    @pl.when(sid == NS - 1)
    def _():
        pltpu.sync_copy(deg2_hbm.at[0, pl.ds(node0, GTAIL)],
                        d0_v.at[pl.ds(0, GTAIL)])
        pltpu.sync_copy(deg2_hbm.at[1, pl.ds(node0, GTAIL)],
                        d1_v.at[pl.ds(0, GTAIL)])
        pltpu.sync_copy(h_hbm.at[pl.ds(node0, GTAIL)],
                        zbuf.at[pl.ds(0, GTAIL)])

    for t in range(ZCHUNK // 16):
        sl = pl.ds(t * 16, 16)
        deg = d0_v[sl] + d1_v[sl] + 1.0
        d0_v[sl] = _rsqrt_nr(deg) * zbuf[sl]

    @pl.when(sid < NS - 1)
    def _():
        pltpu.sync_copy(d0_v, g_sh.at[pl.ds(node0, ZCHUNK)])

    @pl.when(sid == NS - 1)
    def _():
        pltpu.sync_copy(d0_v.at[pl.ds(0, GTAIL)],
                        g_sh.at[pl.ds(node0, GTAIL)])

    _zero_init(zbuf, acc_sh, sid)
    plsc.subcore_barrier()
    pltpu.sync_copy(g_sh.at[pl.ds(0, N_NODES)], g_v.at[pl.ds(0, N_NODES)])

    def row_fn(j, carry):
        base = j * LANES
        for t in range(LANES // 16):
            idx = src_v[pl.ds(base + t * 16, 16)]
            vals_v[pl.ds(base + t * 16, 16)] = plsc.load_gather(g_v, [idx])
        pltpu.async_copy(vals_v.at[pl.ds(base, LANES)],
                         acc_sh.at[dst_v.at[pl.ds(base, LANES)]],
                         sem, add=True)
        return carry

    lax.fori_loop(0, WROWS, row_fn, 0)

    @pl.when(wid < XTRA)
    def _():
        row_fn(WROWS, 0)

    def drain(j, carry):
        pltpu.make_async_copy(vals_v.at[pl.ds(0, LANES)],
                              acc_sh.at[dst_v.at[pl.ds(0, LANES)]],
                              sem).wait()
        return carry

    lax.fori_loop(0, WROWS, drain, 0)

    @pl.when(wid < XTRA)
    def _():
        drain(0, 0)

    plsc.subcore_barrier()

    @pl.when(sid == 0)
    def _():
        pltpu.sync_copy(acc_sh.at[pl.ds(0, N_NODES)], out_hbm.at[cid])


def _matvec_body(x3_ref, w_ref, h_ref):
    h_ref[...] = jnp.sum(x3_ref[...] * w_ref[...], axis=2)


_matvec_call = pl.pallas_call(
    _matvec_body,
    out_shape=jax.ShapeDtypeStruct((NR, 16), jnp.float32),
)


def _combine_body(deg2_ref, h_ref, acc2_ref, out_ref):
    deg = deg2_ref[0] + deg2_ref[1] + 1.0
    dis = lax.rsqrt(deg)
    g = dis * h_ref[...]
    out_ref[...] = dis * (g + acc2_ref[0] + acc2_ref[1])


_combine_call = pl.pallas_call(
    _combine_body,
    out_shape=jax.ShapeDtypeStruct((NR, 16), jnp.float32),
)


def kernel(x, edge_index, W):
    edges = edge_index.astype(jnp.int32)
    x3 = x.reshape(NR, 16, D_FEAT)
    w3 = W.reshape(1, 1, D_FEAT)

    h = _matvec_call(x3, w3)
    deg2 = _hist_kernel(edges)
    acc2 = _msg_kernel(edges, deg2, h.reshape(N_NODES))
    out2d = _combine_call(deg2.reshape(NC, NR, 16), h,
                          acc2.reshape(NC, NR, 16))
    return out2d.reshape(N_NODES)
